# baseline (speedup 1.0000x reference)
import functools

import jax
import jax.numpy as jnp
from jax.experimental import pallas as pl

_ROT0 = (13, 15, 26, 6)
_ROT1 = (17, 29, 16, 24)
_ROUND_ROTS = (_ROT0, _ROT1, _ROT0, _ROT1, _ROT0)
_KS = (0, 1, 0x1BD11BDB)


def _rotl(v, d):
    return (v << jnp.uint32(d)) | (v >> jnp.uint32(32 - d))


def _fill_chunk(inp, mm8, rm8, ctr):
    mm = mm8 != jnp.int8(0)
    rm = rm8 != jnp.int8(0)
    base = jnp.where(mm, jnp.float32(0.0), inp)

    x1 = ctr + jnp.uint32(1)
    x0 = x1
    x1 = x0 ^ _rotl(x1, _ROT0[0])
    for r in _ROT0[1:]:
        x0 = x0 + x1
        x1 = x0 ^ _rotl(x1, r)
    x0 = x0 + jnp.uint32(_KS[1])
    x1 = x1 + jnp.uint32(_KS[2] + 1)
    for g in range(1, 5):
        for r in _ROUND_ROTS[g]:
            x0 = x0 + x1
            x1 = x0 ^ _rotl(x1, r)
        j = g + 1
        if _KS[j % 3]:
            x0 = x0 + jnp.uint32(_KS[j % 3])
        x1 = x1 + jnp.uint32((_KS[(j + 1) % 3] + j) & 0xFFFFFFFF)
    bits = x0 ^ x1

    rv = jnp.where(bits < jnp.uint32(0x80000000), jnp.float32(1.0),
                   jnp.float32(0.0))
    return jnp.where(rm, rv, base)


def _masker_body(inp_ref, mm_ref, rm_ref, out_ref, *, block_rows, ncols,
                 chunk):
    i = pl.program_id(0)
    block_base = i * (block_rows * ncols)
    row = jax.lax.broadcasted_iota(jnp.int32, (chunk, ncols), 0)
    col = jax.lax.broadcasted_iota(jnp.int32, (chunk, ncols), 1)
    local = row * ncols + col

    def step(k, carry):
        sl = pl.ds(k * chunk, chunk)
        ctr = (block_base + k * (chunk * ncols) + local).astype(jnp.uint32)
        out_ref[sl, :] = _fill_chunk(inp_ref[sl, :], mm_ref[sl, :],
                                     rm_ref[sl, :], ctr)
        return carry

    jax.lax.fori_loop(0, block_rows // chunk, step, 0)


def kernel(input, mask_mask, rand_mask):
    nrows, ncols = input.shape
    block_rows = 256
    chunk = 32
    grid = nrows // block_rows
    body = functools.partial(_masker_body, block_rows=block_rows, ncols=ncols,
                             chunk=chunk)
    spec = pl.BlockSpec((block_rows, ncols), lambda i: (i, 0))
    mm8 = mask_mask.view(jnp.int8)
    rm8 = rand_mask.view(jnp.int8)
    return pl.pallas_call(
        body,
        grid=(grid,),
        in_specs=[spec, spec, spec],
        out_specs=spec,
        out_shape=jax.ShapeDtypeStruct(input.shape, input.dtype),
    )(input, mm8, rm8)
